# Initial kernel scaffold; baseline (speedup 1.0000x reference)
import jax
import jax.numpy as jnp
from jax.experimental import pallas as pl


def kernel(x, buffer_img, y, buffer_label, idx, retrieve_idx):
    raise NotImplementedError("write your pallas kernel here")


# speedup vs baseline: 1.5538x; 1.5538x over previous
import functools

import jax
import jax.numpy as jnp
from jax import lax
from jax.experimental import pallas as pl
from jax.experimental.pallas import tpu as pltpu
from jax.experimental.pallas import tpu_sc as plsc

MEM = 50000
B = 1024
D = 3072
NW = 32
BPW = B // NW
CH = 16
RB = 1000
L = 16


def _copy_w_body(idx_col_ref, idx_row_ref, buf_ref, out_ref, w_ref):
    out_ref[...] = buf_ref[...]

    @pl.when(pl.program_id(0) == 0)
    def _():
        eq = idx_col_ref[...] == idx_row_ref[...]
        jj = lax.broadcasted_iota(jnp.int32, (B, B), 1)
        w_ref[...] = jnp.max(jnp.where(eq, jj, -1), axis=1, keepdims=True)


def _tc_copy_w(buf, idx):
    idx_col = idx.reshape(B, 1)
    idx_row = idx.reshape(1, B)
    return pl.pallas_call(
        _copy_w_body,
        grid=(MEM // RB,),
        in_specs=[
            pl.BlockSpec((B, 1), lambda i: (0, 0)),
            pl.BlockSpec((1, B), lambda i: (0, 0)),
            pl.BlockSpec((RB, D), lambda i: (i, 0)),
        ],
        out_specs=[
            pl.BlockSpec((RB, D), lambda i: (i, 0)),
            pl.BlockSpec((B, 1), lambda i: (0, 0)),
        ],
        out_shape=[
            jax.ShapeDtypeStruct((MEM, D), jnp.float32),
            jax.ShapeDtypeStruct((B, 1), jnp.int32),
        ],
    )(idx_col, idx_row, buf)


@functools.cache
def _mesh():
    return plsc.VectorSubcoreMesh(core_axis_name="c", subcore_axis_name="s")


def _sc_scatter_body(x_hbm, idx_hbm, w_hbm, y_hbm, lbl_hbm, buf_ref, out_lbl_hbm,
                     idx_v, w_v, rows_v, lblbuf_v, idxf_v, wf_v, yf_v, sem):
    c = lax.axis_index("c")
    s = lax.axis_index("s")
    wid = s * 2 + c
    base = wid * BPW

    for t in range(BPW // CH):
        off = base + t * CH
        pltpu.sync_copy(idx_hbm.at[pl.ds(off, CH)], idx_v)
        pltpu.sync_copy(w_hbm.at[pl.ds(off, CH)], w_v)
        pltpu.async_copy(x_hbm.at[w_v], rows_v, sem).wait()
        pltpu.async_copy(rows_v, buf_ref.at[idx_v], sem).wait()

    @pl.when(wid == 0)
    def _():
        pltpu.sync_copy(lbl_hbm, lblbuf_v)
        pltpu.sync_copy(idx_hbm, idxf_v)
        pltpu.sync_copy(w_hbm, wf_v)
        pltpu.sync_copy(y_hbm, yf_v)

        def lbody(k, carry):
            iv = idxf_v[pl.ds(k * L, L)]
            wv = wf_v[pl.ds(k * L, L)]
            yv = plsc.load_gather(yf_v, [wv])
            plsc.store_scatter(lblbuf_v, [iv], yv)
            return carry

        lax.fori_loop(0, B // L, lbody, 0)
        pltpu.sync_copy(lblbuf_v, out_lbl_hbm)


@functools.cache
def _sc_scatter_kernel():
    return pl.kernel(
        _sc_scatter_body,
        out_type=jax.ShapeDtypeStruct((MEM,), jnp.int32),
        mesh=_mesh(),
        compiler_params=pltpu.CompilerParams(needs_layout_passes=False),
        scratch_types=[
            pltpu.VMEM((CH,), jnp.int32),
            pltpu.VMEM((CH,), jnp.int32),
            pltpu.VMEM((CH, D), jnp.float32),
            pltpu.VMEM((MEM,), jnp.int32),
            pltpu.VMEM((B,), jnp.int32),
            pltpu.VMEM((B,), jnp.int32),
            pltpu.VMEM((B,), jnp.int32),
            pltpu.SemaphoreType.DMA,
        ],
    )


def _sc_gather_body(buf_hbm, lbl_hbm, ridx_hbm, out_x_hbm, out_y_hbm,
                    ridx_v, rows_v, lblbuf_v, ridxf_v, ry_v, sem):
    c = lax.axis_index("c")
    s = lax.axis_index("s")
    wid = s * 2 + c
    base = wid * BPW

    for t in range(BPW // CH):
        off = base + t * CH
        pltpu.sync_copy(ridx_hbm.at[pl.ds(off, CH)], ridx_v)
        pltpu.async_copy(buf_hbm.at[ridx_v], rows_v, sem).wait()
        pltpu.sync_copy(rows_v, out_x_hbm.at[pl.ds(off, CH)])

    @pl.when(wid == 0)
    def _():
        pltpu.sync_copy(lbl_hbm, lblbuf_v)
        pltpu.sync_copy(ridx_hbm, ridxf_v)

        def lbody(k, carry):
            rv = ridxf_v[pl.ds(k * L, L)]
            ry_v[pl.ds(k * L, L)] = plsc.load_gather(lblbuf_v, [rv])
            return carry

        lax.fori_loop(0, B // L, lbody, 0)
        pltpu.sync_copy(ry_v, out_y_hbm)


@functools.cache
def _sc_gather_kernel():
    return pl.kernel(
        _sc_gather_body,
        out_type=(
            jax.ShapeDtypeStruct((B, D), jnp.float32),
            jax.ShapeDtypeStruct((B,), jnp.int32),
        ),
        mesh=_mesh(),
        compiler_params=pltpu.CompilerParams(needs_layout_passes=False),
        scratch_types=[
            pltpu.VMEM((CH,), jnp.int32),
            pltpu.VMEM((CH, D), jnp.float32),
            pltpu.VMEM((MEM,), jnp.int32),
            pltpu.VMEM((B,), jnp.int32),
            pltpu.VMEM((B,), jnp.int32),
            pltpu.SemaphoreType.DMA,
        ],
    )


def kernel(x, buffer_img, y, buffer_label, idx, retrieve_idx):
    xf = x.reshape(B, D)
    buf = buffer_img.reshape(MEM, D)
    idx = idx.astype(jnp.int32)
    ridx = retrieve_idx.astype(jnp.int32)
    y = y.astype(jnp.int32)

    new_buf, w2 = _tc_copy_w(buf, idx)
    w = w2.reshape(B)

    buf_ref = jax.new_ref(new_buf)
    new_label = _sc_scatter_kernel()(xf, idx, w, y, buffer_label, buf_ref)
    new_buf = buf_ref[...]

    rx, ry = _sc_gather_kernel()(new_buf, new_label, ridx)
    return (
        rx.reshape(B, 3, 32, 32),
        ry,
        new_buf.reshape(MEM, 3, 32, 32),
        new_label,
    )
